# baseline (speedup 1.0000x reference)
import functools

import jax
import jax.numpy as jnp
from jax.experimental import pallas as pl
from jax.experimental.pallas import tpu as pltpu


def _mm_exact(A, B):
    return jnp.sum(A[:, :, None] * B[None, :, :], axis=1)


def _gram_exact(W):
    return jnp.sum(W[:, :, None] * W[:, None, :], axis=0)


def _mm_bt_exact(A, B):
    return jnp.sum(A[:, None, :] * B[None, :, :], axis=2)


def _ortho_body(*refs):
    nw = len(refs) // 2
    w_refs, o_refs = refs[:nw], refs[nw:]
    Ws = [r[...] for r in w_refs]
    cs = [W.shape[1] for W in Ws]
    cmax = max(cs)
    ris = [jax.lax.broadcasted_iota(jnp.int32, (c, c), 0) for c in cs]
    cis = [jax.lax.broadcasted_iota(jnp.int32, (c, c), 1) for c in cs]

    Wbs = [W.astype(jnp.bfloat16).astype(jnp.float32) for W in Ws]
    Gs = []
    for Wb, c, ri, ci in zip(Wbs, cs, ris, cis):
        eye = jnp.where(ri == ci, jnp.float32(1.0), jnp.float32(0.0))
        G = jnp.dot(Wb.T, Wb, preferred_element_type=jnp.float32)
        Gs.append(G + jnp.float32(1e-4) * eye)

    def chol_step(j, carry):
        out = []
        for (A, L), c, ri, ci in zip(carry, cs, ris, cis):
            ajj = jnp.sum(jnp.where((ri == j) & (ci == j), A, 0.0))
            r = jax.lax.rsqrt(ajj)
            r = r * (1.5 - 0.5 * ajj * r * r)
            r = r * (1.5 - 0.5 * ajj * r * r)
            colj = jnp.sum(jnp.where(ci == j, A, 0.0), axis=1, keepdims=True)
            rowj = jnp.sum(jnp.where(ri == j, A, 0.0), axis=0, keepdims=True)
            rge = jax.lax.broadcasted_iota(jnp.int32, (c, 1), 0) >= j
            cge = jax.lax.broadcasted_iota(jnp.int32, (1, c), 1) >= j
            lcol = jnp.where(rge, colj * r, 0.0)
            lrow = jnp.where(cge, rowj * r, 0.0)
            L = jnp.where(ci == j, lcol, L)
            A = A - lcol * lrow
            out.append((A, L))
        return tuple(out)

    carry = tuple((G, jnp.zeros_like(G)) for G in Gs)
    carry = jax.lax.fori_loop(0, cmax, chol_step, carry)
    Ls = [L for _, L in carry]

    Xs = []
    for L, c, ri, ci in zip(Ls, cs, ris, cis):
        diag = jnp.where(ri == ci, L, jnp.float32(1.0))
        rd = jnp.float32(1.0) / diag
        rd = rd * (2.0 - diag * rd)
        rd = rd * (2.0 - diag * rd)
        X = jnp.where(ri == ci, rd, 0.0)
        Xs.append(X)
    eye2 = [jnp.where(ri == ci, jnp.float32(2.0), jnp.float32(0.0))
            for ri, ci in zip(ris, cis)]
    for _ in range(5):
        Xs = [_mm_exact(X, e2 - _mm_exact(L, X))
              for X, L, e2 in zip(Xs, Ls, eye2)]

    for o_ref, Wb, X in zip(o_refs, Wbs, Xs):
        Xb = X.astype(jnp.bfloat16).astype(jnp.float32)
        o_ref[...] = jnp.dot(Wb, Xb.T, preferred_element_type=jnp.float32)


def _ortho_all(*ws):
    return pl.pallas_call(
        _ortho_body,
        out_shape=tuple(jax.ShapeDtypeStruct(w.shape, jnp.float32) for w in ws),
    )(*ws)


def _mm_post_body(flt_ref, a_ref, w_ref, o_ref):
    y = jnp.dot(flt_ref[...], a_ref[...], preferred_element_type=jnp.float32)
    o_ref[...] = jnp.dot(jnp.tanh(y), w_ref[...],
                         preferred_element_type=jnp.float32)


def _mm_tanh_body(flt_ref, a_ref, o_ref):
    y = jnp.dot(flt_ref[...], a_ref[...], preferred_element_type=jnp.float32)
    o_ref[...] = jnp.tanh(y)


def _mm2_post_body(f0_ref, f1_ref, a0_ref, a1_ref, w0_ref, w1_ref,
                   o0_ref, o1_ref):
    y0 = jnp.dot(f0_ref[...], a0_ref[...], preferred_element_type=jnp.float32)
    o0_ref[...] = jnp.dot(jnp.tanh(y0), w0_ref[...],
                          preferred_element_type=jnp.float32)
    y1 = jnp.dot(f1_ref[...], a1_ref[...], preferred_element_type=jnp.float32)
    o1_ref[...] = jnp.dot(jnp.tanh(y1), w1_ref[...],
                          preferred_element_type=jnp.float32)


def _mm2_tanh_body(f0_ref, f1_ref, a0_ref, a1_ref, o0_ref, o1_ref):
    y0 = jnp.dot(f0_ref[...], a0_ref[...], preferred_element_type=jnp.float32)
    o0_ref[...] = jnp.tanh(y0)
    y1 = jnp.dot(f1_ref[...], a1_ref[...], preferred_element_type=jnp.float32)
    o1_ref[...] = jnp.tanh(y1)


def _stream_mm2(flt0, flt1, a0, a1, w0=None, w1=None, bm=200, nbuf=3):
    n, k2 = flt0.shape
    assert n % bm == 0
    grid = (n // bm,)
    row = pl.BlockSpec((bm, k2), lambda i: (i, 0),
                       pipeline_mode=pl.Buffered(buffer_count=nbuf))
    full = lambda x: pl.BlockSpec(x.shape, lambda i: (0, 0))
    in_specs = [row, row, full(a0), full(a1)]
    operands = [flt0, flt1, a0, a1]
    if w0 is None:
        body = _mm2_tanh_body
        kb0, kb1 = a0.shape[1], a1.shape[1]
    else:
        body = _mm2_post_body
        kb0, kb1 = w0.shape[1], w1.shape[1]
        in_specs += [full(w0), full(w1)]
        operands += [w0, w1]
    return pl.pallas_call(
        body,
        grid=grid,
        in_specs=in_specs,
        out_specs=(pl.BlockSpec((bm, kb0), lambda i: (i, 0)),
                   pl.BlockSpec((bm, kb1), lambda i: (i, 0))),
        out_shape=(jax.ShapeDtypeStruct((n, kb0), jnp.float32),
                   jax.ShapeDtypeStruct((n, kb1), jnp.float32)),
        compiler_params=pltpu.CompilerParams(
            dimension_semantics=("parallel",)),
    )(*operands)


def _mms_body(*refs, s, bm, post):
    if post:
        f_refs, (a_ref, w_ref, o_ref) = refs[:s], refs[s:]
    else:
        f_refs, (a_ref, o_ref) = refs[:s], refs[s:]
    a = a_ref[...]
    for j in range(s):
        y = jnp.dot(f_refs[j][...], a, preferred_element_type=jnp.float32)
        h = jnp.tanh(y)
        if post:
            h = jnp.dot(h, w_ref[...], preferred_element_type=jnp.float32)
        o_ref[j * bm:(j + 1) * bm, :] = h


def _stream_mm(flt, a, w=None, bm=40, s=10):
    n, k2 = flt.shape
    per = bm * s
    assert n % per == 0
    ka = a.shape[1]
    grid = (n // per,)

    def idx(j):
        return lambda i: (s * i + j, 0)

    in_specs = [pl.BlockSpec((bm, k2), idx(j)) for j in range(s)]
    in_specs.append(pl.BlockSpec((k2, ka), lambda i: (0, 0)))
    operands = [flt] * s + [a]
    if w is None:
        kb = ka
    else:
        kb = w.shape[1]
        in_specs.append(pl.BlockSpec(w.shape, lambda i: (0, 0)))
        operands.append(w)
    body = functools.partial(_mms_body, s=s, bm=bm, post=w is not None)
    return pl.pallas_call(
        body,
        grid=grid,
        in_specs=in_specs,
        out_specs=pl.BlockSpec((per, kb), lambda i: (i, 0)),
        out_shape=jax.ShapeDtypeStruct((n, kb), jnp.float32),
        compiler_params=pltpu.CompilerParams(
            dimension_semantics=("parallel",)),
    )(*operands)


def _proj_body(x_ref, w_ref, o_ref):
    o_ref[...] = jnp.dot(x_ref[...], w_ref[...],
                         preferred_element_type=jnp.float32)


def _proj(x, w):
    return pl.pallas_call(
        _proj_body,
        out_shape=jax.ShapeDtypeStruct((x.shape[0], w.shape[1]), jnp.float32),
    )(x, w)


def _fuse_body(h0_ref, h1_ref, u0_ref, u1_ref, h0c_ref, h1c_ref, hid_ref):
    h0c = h0_ref[...] - jnp.mean(h0_ref[...], axis=0, keepdims=True)
    h1c = h1_ref[...] - jnp.mean(h1_ref[...], axis=0, keepdims=True)
    h0c_ref[...] = h0c
    h1c_ref[...] = h1c
    hid_ref[...] = (jnp.dot(h0c, u0_ref[...], preferred_element_type=jnp.float32)
                    + jnp.dot(h1c, u1_ref[...], preferred_element_type=jnp.float32))


def _fuse(h0, h1, u0, u1):
    n, c = h0.shape
    return pl.pallas_call(
        _fuse_body,
        out_shape=(
            jax.ShapeDtypeStruct((n, c), jnp.float32),
            jax.ShapeDtypeStruct((n, c), jnp.float32),
            jax.ShapeDtypeStruct((n, u0.shape[1]), jnp.float32),
        ),
    )(h0, h1, u0, u1)


def kernel(feat0, feat1, flt0, flt1, flt_f, gc1_w0, gc2_w0, gc1_w1, gc2_w1,
           fus_w0, fus_w1):
    w10, w20, w11, w21, u0, u1 = _ortho_all(
        gc1_w0, gc2_w0, gc1_w1, gc2_w1, fus_w0, fus_w1)

    a0 = _proj(feat0, w10)
    a1 = _proj(feat1, w11)
    b0 = _stream_mm(flt0, a0, w=w20)
    b1 = _stream_mm(flt1, a1, w=w21)
    h0 = _stream_mm(flt0, b0)
    h1 = _stream_mm(flt1, b1)
    h0c, h1c, hidden = _fuse(h0, h1, u0, u1)
    common = _stream_mm(flt_f, hidden)
    return (common, h0c, h1c)


# speedup vs baseline: 1.0531x; 1.0531x over previous
import functools

import jax
import jax.numpy as jnp
from jax.experimental import pallas as pl
from jax.experimental.pallas import tpu as pltpu


def _mm_exact(A, B):
    return jnp.sum(A[:, :, None] * B[None, :, :], axis=1)


def _gram_exact(W):
    return jnp.sum(W[:, :, None] * W[:, None, :], axis=0)


def _mm_bt_exact(A, B):
    return jnp.sum(A[:, None, :] * B[None, :, :], axis=2)


def _ortho_body(*refs):
    nw = len(refs) // 2
    w_refs, o_refs = refs[:nw], refs[nw:]
    Ws = [r[...] for r in w_refs]
    cs = [W.shape[1] for W in Ws]
    cmax = max(cs)
    ris = [jax.lax.broadcasted_iota(jnp.int32, (c, c), 0) for c in cs]
    cis = [jax.lax.broadcasted_iota(jnp.int32, (c, c), 1) for c in cs]

    Wbs = [W.astype(jnp.bfloat16).astype(jnp.float32) for W in Ws]
    Gs = []
    for Wb, c, ri, ci in zip(Wbs, cs, ris, cis):
        eye = jnp.where(ri == ci, jnp.float32(1.0), jnp.float32(0.0))
        G = jnp.dot(Wb.T, Wb, preferred_element_type=jnp.float32)
        Gs.append(G + jnp.float32(1e-4) * eye)

    def chol_step(j, carry):
        out = []
        for (A, L), c, ri, ci in zip(carry, cs, ris, cis):
            ajj = jnp.sum(jnp.where((ri == j) & (ci == j), A, 0.0))
            r = jax.lax.rsqrt(ajj)
            r = r * (1.5 - 0.5 * ajj * r * r)
            r = r * (1.5 - 0.5 * ajj * r * r)
            colj = jnp.sum(jnp.where(ci == j, A, 0.0), axis=1, keepdims=True)
            rowj = jnp.sum(jnp.where(ri == j, A, 0.0), axis=0, keepdims=True)
            rge = jax.lax.broadcasted_iota(jnp.int32, (c, 1), 0) >= j
            cge = jax.lax.broadcasted_iota(jnp.int32, (1, c), 1) >= j
            lcol = jnp.where(rge, colj * r, 0.0)
            lrow = jnp.where(cge, rowj * r, 0.0)
            L = jnp.where(ci == j, lcol, L)
            A = A - lcol * lrow
            out.append((A, L))
        return tuple(out)

    carry = tuple((G, jnp.zeros_like(G)) for G in Gs)
    carry = jax.lax.fori_loop(0, cmax, chol_step, carry)
    Ls = [L for _, L in carry]

    Xs = []
    for L, c, ri, ci in zip(Ls, cs, ris, cis):
        diag = jnp.where(ri == ci, L, jnp.float32(1.0))
        rd = jnp.float32(1.0) / diag
        rd = rd * (2.0 - diag * rd)
        rd = rd * (2.0 - diag * rd)
        X = jnp.where(ri == ci, rd, 0.0)
        Xs.append(X)
    eye2 = [jnp.where(ri == ci, jnp.float32(2.0), jnp.float32(0.0))
            for ri, ci in zip(ris, cis)]
    for _ in range(5):
        Xs = [_mm_exact(X, e2 - _mm_exact(L, X))
              for X, L, e2 in zip(Xs, Ls, eye2)]

    for o_ref, Wb, X in zip(o_refs, Wbs, Xs):
        Xb = X.astype(jnp.bfloat16).astype(jnp.float32)
        o_ref[...] = jnp.dot(Wb, Xb.T, preferred_element_type=jnp.float32)


def _ortho_all(*ws):
    return pl.pallas_call(
        _ortho_body,
        out_shape=tuple(jax.ShapeDtypeStruct(w.shape, jnp.float32) for w in ws),
    )(*ws)


def _mm_post_body(flt_ref, a_ref, w_ref, o_ref):
    y = jnp.dot(flt_ref[...], a_ref[...], preferred_element_type=jnp.float32)
    o_ref[...] = jnp.dot(jnp.tanh(y), w_ref[...],
                         preferred_element_type=jnp.float32)


def _mm_tanh_body(flt_ref, a_ref, o_ref):
    y = jnp.dot(flt_ref[...], a_ref[...], preferred_element_type=jnp.float32)
    o_ref[...] = jnp.tanh(y)


def _mm2_post_body(f0_ref, f1_ref, a0_ref, a1_ref, w0_ref, w1_ref,
                   o0_ref, o1_ref):
    y0 = jnp.dot(f0_ref[...], a0_ref[...], preferred_element_type=jnp.float32)
    o0_ref[...] = jnp.dot(jnp.tanh(y0), w0_ref[...],
                          preferred_element_type=jnp.float32)
    y1 = jnp.dot(f1_ref[...], a1_ref[...], preferred_element_type=jnp.float32)
    o1_ref[...] = jnp.dot(jnp.tanh(y1), w1_ref[...],
                          preferred_element_type=jnp.float32)


def _mm2_tanh_body(f0_ref, f1_ref, a0_ref, a1_ref, o0_ref, o1_ref):
    y0 = jnp.dot(f0_ref[...], a0_ref[...], preferred_element_type=jnp.float32)
    o0_ref[...] = jnp.tanh(y0)
    y1 = jnp.dot(f1_ref[...], a1_ref[...], preferred_element_type=jnp.float32)
    o1_ref[...] = jnp.tanh(y1)


def _stream_mm2(flt0, flt1, a0, a1, w0=None, w1=None, bm=200, nbuf=3):
    n, k2 = flt0.shape
    assert n % bm == 0
    grid = (n // bm,)
    row = pl.BlockSpec((bm, k2), lambda i: (i, 0),
                       pipeline_mode=pl.Buffered(buffer_count=nbuf))
    full = lambda x: pl.BlockSpec(x.shape, lambda i: (0, 0))
    in_specs = [row, row, full(a0), full(a1)]
    operands = [flt0, flt1, a0, a1]
    if w0 is None:
        body = _mm2_tanh_body
        kb0, kb1 = a0.shape[1], a1.shape[1]
    else:
        body = _mm2_post_body
        kb0, kb1 = w0.shape[1], w1.shape[1]
        in_specs += [full(w0), full(w1)]
        operands += [w0, w1]
    return pl.pallas_call(
        body,
        grid=grid,
        in_specs=in_specs,
        out_specs=(pl.BlockSpec((bm, kb0), lambda i: (i, 0)),
                   pl.BlockSpec((bm, kb1), lambda i: (i, 0))),
        out_shape=(jax.ShapeDtypeStruct((n, kb0), jnp.float32),
                   jax.ShapeDtypeStruct((n, kb1), jnp.float32)),
        compiler_params=pltpu.CompilerParams(
            dimension_semantics=("parallel",)),
    )(*operands)


def _mms_body(*refs, s, bm, post):
    if post:
        f_refs, (a_ref, w_ref, o_ref) = refs[:s], refs[s:]
    else:
        f_refs, (a_ref, o_ref) = refs[:s], refs[s:]
    a = a_ref[...]
    for j in range(s):
        y = jnp.dot(f_refs[j][...], a, preferred_element_type=jnp.float32)
        h = jnp.tanh(y)
        if post:
            h = jnp.dot(h, w_ref[...], preferred_element_type=jnp.float32)
        o_ref[j * bm:(j + 1) * bm, :] = h


def _stream_mm(flt, a, w=None, bm=80, s=5):
    n, k2 = flt.shape
    per = bm * s
    assert n % per == 0
    ka = a.shape[1]
    grid = (n // per,)

    def idx(j):
        return lambda i: (s * i + j, 0)

    in_specs = [pl.BlockSpec((bm, k2), idx(j)) for j in range(s)]
    in_specs.append(pl.BlockSpec((k2, ka), lambda i: (0, 0)))
    operands = [flt] * s + [a]
    if w is None:
        kb = ka
    else:
        kb = w.shape[1]
        in_specs.append(pl.BlockSpec(w.shape, lambda i: (0, 0)))
        operands.append(w)
    body = functools.partial(_mms_body, s=s, bm=bm, post=w is not None)
    return pl.pallas_call(
        body,
        grid=grid,
        in_specs=in_specs,
        out_specs=pl.BlockSpec((per, kb), lambda i: (i, 0)),
        out_shape=jax.ShapeDtypeStruct((n, kb), jnp.float32),
        compiler_params=pltpu.CompilerParams(
            dimension_semantics=("parallel",)),
    )(*operands)


def _l1_body(*refs, s, bm):
    f_refs = refs[:s]
    feat_ref, w1_ref, w2_ref, o_ref, a_ref = refs[s:]

    @pl.when(pl.program_id(0) == 0)
    def _():
        a_ref[...] = jnp.dot(feat_ref[...], w1_ref[...],
                             preferred_element_type=jnp.float32)

    a = a_ref[...]
    w2 = w2_ref[...]
    for j in range(s):
        y = jnp.dot(f_refs[j][...], a, preferred_element_type=jnp.float32)
        o_ref[j * bm:(j + 1) * bm, :] = jnp.dot(
            jnp.tanh(y), w2, preferred_element_type=jnp.float32)


def _l1_stream(flt, feat, w1, w2, bm=80, s=5):
    n, k2 = flt.shape
    per = bm * s
    assert n % per == 0
    ka = w1.shape[1]
    kb = w2.shape[1]
    grid = (n // per,)

    def idx(j):
        return lambda i: (s * i + j, 0)

    in_specs = [pl.BlockSpec((bm, k2), idx(j)) for j in range(s)]
    in_specs += [
        pl.BlockSpec(feat.shape, lambda i: (0, 0)),
        pl.BlockSpec(w1.shape, lambda i: (0, 0)),
        pl.BlockSpec(w2.shape, lambda i: (0, 0)),
    ]
    return pl.pallas_call(
        functools.partial(_l1_body, s=s, bm=bm),
        grid=grid,
        in_specs=in_specs,
        out_specs=pl.BlockSpec((per, kb), lambda i: (i, 0)),
        out_shape=jax.ShapeDtypeStruct((n, kb), jnp.float32),
        scratch_shapes=[pltpu.VMEM((n, ka), jnp.float32)],
        compiler_params=pltpu.CompilerParams(
            dimension_semantics=("arbitrary",)),
    )(*([flt] * s + [feat, w1, w2]))


def _proj_body(x_ref, w_ref, o_ref):
    o_ref[...] = jnp.dot(x_ref[...], w_ref[...],
                         preferred_element_type=jnp.float32)


def _proj(x, w):
    return pl.pallas_call(
        _proj_body,
        out_shape=jax.ShapeDtypeStruct((x.shape[0], w.shape[1]), jnp.float32),
    )(x, w)


def _fuse_body(h0_ref, h1_ref, u0_ref, u1_ref, h0c_ref, h1c_ref, hid_ref):
    h0c = h0_ref[...] - jnp.mean(h0_ref[...], axis=0, keepdims=True)
    h1c = h1_ref[...] - jnp.mean(h1_ref[...], axis=0, keepdims=True)
    h0c_ref[...] = h0c
    h1c_ref[...] = h1c
    hid_ref[...] = (jnp.dot(h0c, u0_ref[...], preferred_element_type=jnp.float32)
                    + jnp.dot(h1c, u1_ref[...], preferred_element_type=jnp.float32))


def _fuse(h0, h1, u0, u1):
    n, c = h0.shape
    return pl.pallas_call(
        _fuse_body,
        out_shape=(
            jax.ShapeDtypeStruct((n, c), jnp.float32),
            jax.ShapeDtypeStruct((n, c), jnp.float32),
            jax.ShapeDtypeStruct((n, u0.shape[1]), jnp.float32),
        ),
    )(h0, h1, u0, u1)


def kernel(feat0, feat1, flt0, flt1, flt_f, gc1_w0, gc2_w0, gc1_w1, gc2_w1,
           fus_w0, fus_w1):
    w10, w20, w11, w21, u0, u1 = _ortho_all(
        gc1_w0, gc2_w0, gc1_w1, gc2_w1, fus_w0, fus_w1)

    b0 = _l1_stream(flt0, feat0, w10, w20)
    b1 = _l1_stream(flt1, feat1, w11, w21)
    h0 = _stream_mm(flt0, b0)
    h1 = _stream_mm(flt1, b1)
    h0c, h1c, hidden = _fuse(h0, h1, u0, u1)
    common = _stream_mm(flt_f, hidden)
    return (common, h0c, h1c)
